# Initial kernel scaffold; baseline (speedup 1.0000x reference)
import jax
import jax.numpy as jnp
from jax.experimental import pallas as pl


def kernel(x, conn, W1, Wg1, bg1, Wg2, bg2):
    raise NotImplementedError("write your pallas kernel here")


# speedup vs baseline: 1.3486x; 1.3486x over previous
import functools

import jax
import jax.numpy as jnp
from jax import lax
from jax.experimental import pallas as pl
from jax.experimental.pallas import tpu as pltpu
from jax.experimental.pallas import tpu_sc as plsc

N = 10000
K = 32
D = 128
LANES = 8

NW = 32
N_PAD = 10240
PER_TILE = N_PAD // NW
GB = 4
NG = PER_TILE // GB

_INV_K = 1.0 / K


def _mm_at(a, b):
    return lax.dot_general(a, b, (((1,), (1,)), ((), ())),
                           preferred_element_type=jnp.float32)


def _stage_a_body(x_ref, w1_ref, wg1_ref, o_ref):
    h = jnp.maximum(_mm_at(x_ref[...], w1_ref[...]), 0.0)
    o_ref[...] = _mm_at(h, wg1_ref[...])


def _stage_a(x, w1, wg1):
    blk = 1000
    grid = N // blk
    return pl.pallas_call(
        _stage_a_body,
        grid=(grid,),
        in_specs=[
            pl.BlockSpec((blk, D), lambda i: (i, 0)),
            pl.BlockSpec((D, D), lambda i: (0, 0)),
            pl.BlockSpec((D, D), lambda i: (0, 0)),
        ],
        out_specs=pl.BlockSpec((blk, D), lambda i: (i, 0)),
        out_shape=jax.ShapeDtypeStruct((N, D), jnp.float32),
    )(x, w1, wg1)


def _stage_b_body(s_ref, bg_ref, wg2_ref, o_ref):
    a = s_ref[...] * _INV_K + bg_ref[...]
    h = jnp.maximum(a, 0.0)
    o_ref[...] = _mm_at(h, wg2_ref[...])


def _stage_b(s1, bg1, wg2):
    blk = 1024
    grid = N_PAD // blk
    return pl.pallas_call(
        _stage_b_body,
        grid=(grid,),
        in_specs=[
            pl.BlockSpec((blk, D), lambda i: (i, 0)),
            pl.BlockSpec((1, D), lambda i: (0, 0)),
            pl.BlockSpec((D, D), lambda i: (0, 0)),
        ],
        out_specs=pl.BlockSpec((blk, D), lambda i: (i, 0)),
        out_shape=jax.ShapeDtypeStruct((N_PAD, D), jnp.float32),
    )(s1, bg1.reshape(1, D), wg2)


def _stage_c_body(s_ref, bg_ref, o_ref):
    a = s_ref[...] * _INV_K + bg_ref[...]
    nrm = jnp.sqrt(jnp.sum(a * a, axis=1, keepdims=True))
    o_ref[...] = a / nrm


def _stage_c(s2, bg2):
    blk = 1000
    grid = N // blk
    return pl.pallas_call(
        _stage_c_body,
        grid=(grid,),
        in_specs=[
            pl.BlockSpec((blk, D), lambda i: (i, 0)),
            pl.BlockSpec((1, D), lambda i: (0, 0)),
        ],
        out_specs=pl.BlockSpec((blk, D), lambda i: (i, 0)),
        out_shape=jax.ShapeDtypeStruct((N, D), jnp.float32),
    )(s2, bg2.reshape(1, D))


def _sc_body(table_hbm, idx_hbm, out_hbm, idx_v, buf_v, out_v, sem):
    wid = lax.axis_index("s") * 2 + lax.axis_index("c")
    base = wid * PER_TILE
    pltpu.sync_copy(idx_hbm.at[wid], idx_v)

    @pl.loop(0, NG)
    def _(g):
        pltpu.async_copy(table_hbm.at[idx_v.at[g]], buf_v, sem).wait()
        for n in range(GB):
            row0 = n * K

            def kbody(k, acc):
                return tuple(
                    acc[c] + buf_v[row0 + k, pl.ds(c * 16, 16)]
                    for c in range(LANES))

            init = tuple(
                buf_v[row0, pl.ds(c * 16, 16)] for c in range(LANES))
            acc = lax.fori_loop(1, K, kbody, init)
            for c in range(LANES):
                out_v[g * GB + n, pl.ds(c * 16, 16)] = acc[c]

    pltpu.sync_copy(out_v, out_hbm.at[pl.ds(base, PER_TILE)])


@functools.cache
def _sc_gather_sum():
    return pl.kernel(
        _sc_body,
        out_type=jax.ShapeDtypeStruct((N_PAD, D), jnp.float32),
        mesh=plsc.VectorSubcoreMesh(core_axis_name="c", subcore_axis_name="s",
                                    num_cores=2, num_subcores=16),
        scratch_types=[
            pltpu.VMEM((NG, GB * K), jnp.int32),
            pltpu.VMEM((GB * K, D), jnp.float32),
            pltpu.VMEM((PER_TILE, D), jnp.float32),
            pltpu.SemaphoreType.DMA,
        ],
    )


def kernel(x, conn, W1, Wg1, bg1, Wg2, bg2):
    conn32 = conn.astype(jnp.int32)
    idx_arr = jnp.pad(conn32, ((0, N_PAD - N), (0, 0))).reshape(NW, NG, GB * K)

    sc_gather = _sc_gather_sum()
    g1 = _stage_a(x, W1, Wg1)
    s1 = sc_gather(g1, idx_arr)
    g2 = _stage_b(s1, bg1, Wg2)
    s2 = sc_gather(g2, idx_arr)
    return _stage_c(s2[:N], bg2)


# speedup vs baseline: 1.7957x; 1.3316x over previous
import functools

import jax
import jax.numpy as jnp
from jax import lax
from jax.experimental import pallas as pl
from jax.experimental.pallas import tpu as pltpu
from jax.experimental.pallas import tpu_sc as plsc

N = 10000
K = 32
D = 128
LANES = 8

NW = 32
N_PAD = 10240
PER_TILE = N_PAD // NW
BN = 80
NBLK = PER_TILE // BN

_INV_K = 1.0 / K


def _mm_at(a, b):
    return lax.dot_general(a, b, (((1,), (1,)), ((), ())),
                           preferred_element_type=jnp.float32)


def _stage_a_body(x_ref, w1_ref, wg1_ref, o_ref):
    h = jnp.maximum(_mm_at(x_ref[...], w1_ref[...]), 0.0)
    o_ref[...] = _mm_at(h, wg1_ref[...])


def _stage_a(x, w1, wg1):
    blk = 1000
    grid = N // blk
    return pl.pallas_call(
        _stage_a_body,
        grid=(grid,),
        in_specs=[
            pl.BlockSpec((blk, D), lambda i: (i, 0)),
            pl.BlockSpec((D, D), lambda i: (0, 0)),
            pl.BlockSpec((D, D), lambda i: (0, 0)),
        ],
        out_specs=pl.BlockSpec((blk, D), lambda i: (i, 0)),
        out_shape=jax.ShapeDtypeStruct((N, D), jnp.float32),
    )(x, w1, wg1)


def _stage_b_body(s_ref, bg_ref, wg2_ref, o_ref):
    a = s_ref[...] * _INV_K + bg_ref[...]
    h = jnp.maximum(a, 0.0)
    o_ref[...] = _mm_at(h, wg2_ref[...])


def _stage_b(s1, bg1, wg2):
    blk = 1024
    grid = N_PAD // blk
    return pl.pallas_call(
        _stage_b_body,
        grid=(grid,),
        in_specs=[
            pl.BlockSpec((blk, D), lambda i: (i, 0)),
            pl.BlockSpec((1, D), lambda i: (0, 0)),
            pl.BlockSpec((D, D), lambda i: (0, 0)),
        ],
        out_specs=pl.BlockSpec((blk, D), lambda i: (i, 0)),
        out_shape=jax.ShapeDtypeStruct((N_PAD, D), jnp.float32),
    )(s1, bg1.reshape(1, D), wg2)


def _stage_c_body(s_ref, bg_ref, o_ref):
    a = s_ref[...] * _INV_K + bg_ref[...]
    nrm = jnp.sqrt(jnp.sum(a * a, axis=1, keepdims=True))
    o_ref[...] = a / nrm


def _stage_c(s2, bg2):
    blk = 1000
    grid = N // blk
    return pl.pallas_call(
        _stage_c_body,
        grid=(grid,),
        in_specs=[
            pl.BlockSpec((blk, D), lambda i: (i, 0)),
            pl.BlockSpec((1, D), lambda i: (0, 0)),
        ],
        out_specs=pl.BlockSpec((blk, D), lambda i: (i, 0)),
        out_shape=jax.ShapeDtypeStruct((N, D), jnp.float32),
    )(s2, bg2.reshape(1, D))


def _sc_body(table_hbm, connt_hbm, out_hbm,
             idx_v, acc0, acc1, gsem0, gsem1, osem0, osem1):
    wid = lax.axis_index("s") * 2 + lax.axis_index("c")
    base = wid * PER_TILE
    pltpu.sync_copy(connt_hbm.at[:, wid], idx_v)
    accs = (acc0, acc1)
    gsems = (gsem0, gsem1)
    osems = (osem0, osem1)

    def zero(accb):
        @pl.loop(0, BN)
        def _(r):
            for c in range(LANES):
                accb[r, pl.ds(c * 16, 16)] = jnp.zeros((16,), jnp.float32)

    def issue(blk, slot):
        @pl.loop(0, K)
        def _(k):
            pltpu.async_copy(
                table_hbm.at[idx_v.at[k, blk]],
                accs[slot], gsems[slot], add=True)

    def drain(slot):
        @pl.loop(0, K)
        def _(k):
            pltpu.make_async_copy(
                table_hbm.at[idx_v.at[0, 0]],
                accs[slot], gsems[slot]).wait()

    zero(acc0)
    zero(acc1)
    issue(0, 0)
    issue(1, 1)
    for b in range(NBLK):
        slot = b % 2
        drain(slot)
        pltpu.async_copy(
            accs[slot], out_hbm.at[pl.ds(base + b * BN, BN)],
            osems[slot]).wait()
        if b + 2 < NBLK:
            zero(accs[slot])
            issue(b + 2, slot)


@functools.cache
def _sc_gather_sum():
    return pl.kernel(
        _sc_body,
        out_type=jax.ShapeDtypeStruct((N_PAD, D), jnp.float32),
        mesh=plsc.VectorSubcoreMesh(core_axis_name="c", subcore_axis_name="s",
                                    num_cores=2, num_subcores=16),
        scratch_types=[
            pltpu.VMEM((K, NBLK, BN), jnp.int32),
            pltpu.VMEM((BN, D), jnp.float32),
            pltpu.VMEM((BN, D), jnp.float32),
            pltpu.SemaphoreType.DMA,
            pltpu.SemaphoreType.DMA,
            pltpu.SemaphoreType.DMA,
            pltpu.SemaphoreType.DMA,
        ],
    )


def kernel(x, conn, W1, Wg1, bg1, Wg2, bg2):
    conn32 = conn.astype(jnp.int32)
    conn_t = jnp.pad(conn32.T, ((0, 0), (0, N_PAD - N))).reshape(
        K, NW, NBLK, BN)

    sc_gather = _sc_gather_sum()
    g1 = _stage_a(x, W1, Wg1)
    s1 = sc_gather(g1, conn_t)
    g2 = _stage_b(s1, bg1, Wg2)
    s2 = sc_gather(g2, conn_t)
    return _stage_c(s2[:N], bg2)


# speedup vs baseline: 1.8017x; 1.0033x over previous
import functools

import jax
import jax.numpy as jnp
from jax import lax
from jax.experimental import pallas as pl
from jax.experimental.pallas import tpu as pltpu
from jax.experimental.pallas import tpu_sc as plsc

N = 10000
K = 32
D = 128
LANES = 8

NS = 16
BN = 80
NBLK0 = 7
NBLK1 = 1
NBLK_TOT = NBLK0 + NBLK1
N_PAD = NS * NBLK_TOT * BN
TOTBLK = N_PAD // BN

_INV_K = 1.0 / K


def _mm_at(a, b):
    return lax.dot_general(a, b, (((1,), (1,)), ((), ())),
                           preferred_element_type=jnp.float32)


def _stage_a_body(x_ref, w1_ref, wg1_ref, o_ref):
    h = jnp.maximum(_mm_at(x_ref[...], w1_ref[...]), 0.0)
    o_ref[...] = _mm_at(h, wg1_ref[...])


def _stage_a(x, w1, wg1):
    blk = 1000
    grid = N // blk
    return pl.pallas_call(
        _stage_a_body,
        grid=(grid,),
        in_specs=[
            pl.BlockSpec((blk, D), lambda i: (i, 0)),
            pl.BlockSpec((D, D), lambda i: (0, 0)),
            pl.BlockSpec((D, D), lambda i: (0, 0)),
        ],
        out_specs=pl.BlockSpec((blk, D), lambda i: (i, 0)),
        out_shape=jax.ShapeDtypeStruct((N, D), jnp.float32),
    )(x, w1, wg1)


def _stage_b_body(s_ref, bg_ref, wg2_ref, o_ref):
    a = s_ref[...] * _INV_K + bg_ref[...]
    h = jnp.maximum(a, 0.0)
    o_ref[...] = _mm_at(h, wg2_ref[...])


def _stage_b(s1, bg1, wg2):
    blk = 1024
    grid = N_PAD // blk
    return pl.pallas_call(
        _stage_b_body,
        grid=(grid,),
        in_specs=[
            pl.BlockSpec((blk, D), lambda i: (i, 0)),
            pl.BlockSpec((1, D), lambda i: (0, 0)),
            pl.BlockSpec((D, D), lambda i: (0, 0)),
        ],
        out_specs=pl.BlockSpec((blk, D), lambda i: (i, 0)),
        out_shape=jax.ShapeDtypeStruct((N_PAD, D), jnp.float32),
    )(s1, bg1.reshape(1, D), wg2)


def _stage_c_body(s_ref, bg_ref, o_ref):
    a = s_ref[...] * _INV_K + bg_ref[...]
    nrm = jnp.sqrt(jnp.sum(a * a, axis=1, keepdims=True))
    o_ref[...] = a / nrm


def _stage_c(s2, bg2):
    blk = 1000
    grid = N // blk
    return pl.pallas_call(
        _stage_c_body,
        grid=(grid,),
        in_specs=[
            pl.BlockSpec((blk, D), lambda i: (i, 0)),
            pl.BlockSpec((1, D), lambda i: (0, 0)),
        ],
        out_specs=pl.BlockSpec((blk, D), lambda i: (i, 0)),
        out_shape=jax.ShapeDtypeStruct((N, D), jnp.float32),
    )(s2, bg2.reshape(1, D))


def _sc_body(table_hbm, connt0_hbm, connt1_hbm, out_hbm,
             idx_v, acc0, acc1, gsem0, gsem1, osem0, osem1):
    cid = lax.axis_index("c")
    sid = lax.axis_index("s")
    accs = (acc0, acc1)
    gsems = (gsem0, gsem1)
    osems = (osem0, osem1)

    def zero(accb):
        @pl.loop(0, BN)
        def _(r):
            for c in range(LANES):
                accb[r, pl.ds(c * 16, 16)] = jnp.zeros((16,), jnp.float32)

    def issue(b, slot):
        @pl.loop(0, K)
        def _(k):
            pltpu.async_copy(
                table_hbm.at[idx_v.at[k, b]],
                accs[slot], gsems[slot], add=True)

    def drain(slot):
        @pl.loop(0, K)
        def _(k):
            pltpu.make_async_copy(
                table_hbm.at[idx_v.at[0, 0]],
                accs[slot], gsems[slot]).wait()

    def run(nblk, first_blk):
        zero(acc0)
        issue(0, 0)
        if nblk > 1:
            zero(acc1)
            issue(1, 1)
        for b in range(nblk):
            slot = b % 2
            drain(slot)
            pltpu.async_copy(
                accs[slot],
                out_hbm.at[pl.ds((first_blk + b) * BN, BN)],
                osems[slot]).wait()
            if b + 2 < nblk:
                zero(accs[slot])
                issue(b + 2, slot)

    @pl.when(cid == 0)
    def _():
        pltpu.sync_copy(connt0_hbm.at[:, sid], idx_v)
        run(NBLK0, sid * NBLK0)

    @pl.when(cid == 1)
    def _():
        pltpu.sync_copy(connt1_hbm.at[:, sid], idx_v.at[:, 0])
        run(NBLK1, NS * NBLK0 + sid)


@functools.cache
def _sc_gather_sum():
    return pl.kernel(
        _sc_body,
        out_type=jax.ShapeDtypeStruct((N_PAD, D), jnp.float32),
        mesh=plsc.VectorSubcoreMesh(core_axis_name="c", subcore_axis_name="s",
                                    num_cores=2, num_subcores=16),
        scratch_types=[
            pltpu.VMEM((K, NBLK0, BN), jnp.int32),
            pltpu.VMEM((BN, D), jnp.float32),
            pltpu.VMEM((BN, D), jnp.float32),
            pltpu.SemaphoreType.DMA,
            pltpu.SemaphoreType.DMA,
            pltpu.SemaphoreType.DMA,
            pltpu.SemaphoreType.DMA,
        ],
    )


def kernel(x, conn, W1, Wg1, bg1, Wg2, bg2):
    conn32 = conn.astype(jnp.int32)
    conn_t = jnp.pad(conn32.T, ((0, 0), (0, N_PAD - N)))
    n0 = NS * NBLK0 * BN
    connt0 = conn_t[:, :n0].reshape(K, NS, NBLK0, BN)
    connt1 = conn_t[:, n0:].reshape(K, NS, BN)

    sc_gather = _sc_gather_sum()
    g1 = _stage_a(x, W1, Wg1)
    s1 = sc_gather(g1, connt0, connt1)
    g2 = _stage_b(s1, bg1, Wg2)
    s2 = sc_gather(g2, connt0, connt1)
    return _stage_c(s2[:N], bg2)
